# baseline (speedup 1.0000x reference)
import functools

import jax
import jax.numpy as jnp
from jax import lax
from jax.experimental import pallas as pl
from jax.experimental.pallas import tpu as pltpu
from jax.experimental.pallas import tpu_sc as plsc

_NUM_EXPERTS = 64
_TOP_K = 2
_B, _S, _D = 4, 8192, 768
_LANES = _D // 16

_HEAD_S = 7680
_TAIL_S = _S - _HEAD_S
_S_BLK = 256
_N_BLK = _HEAD_S // _S_BLK

_NC, _NS = 2, 16
_NW = _NC * _NS
_JPB = 8
_RPW = _TAIL_S // _JPB
_HLANES = _LANES // 2
_CH = 32
_NCH = _RPW // _CH


def _tc_head_kernel(x_ref, out_ref):
    step = pl.program_id(0)

    @pl.when(step == 0)
    def _init():
        out_ref[...] = jnp.zeros_like(out_ref)

    out_ref[...] += jnp.sum(x_ref[...], axis=1)


_sc_mesh = plsc.VectorSubcoreMesh(core_axis_name="c", subcore_axis_name="s")


@functools.partial(
    pl.kernel,
    mesh=_sc_mesh,
    out_type=jax.ShapeDtypeStruct((_NW, _D), jnp.float32),
    scratch_types=[
        pltpu.VMEM((2, _CH, _D), jnp.float32),
        pltpu.VMEM((_D,), jnp.float32),
        pltpu.VMEM((2, _CH), jnp.int32),
        pltpu.SemaphoreType.DMA,
        pltpu.SemaphoreType.DMA,
    ],
    cost_estimate=pl.CostEstimate(
        flops=_B * _TAIL_S * _D,
        bytes_accessed=_B * _TAIL_S * _D * 4,
        transcendentals=0,
    ),
)
def _sc_tail_kernel(x_hbm, out_hbm, buf, accv, idxv, sem0, sem1):
    cid = lax.axis_index("c")
    sid = lax.axis_index("s")
    wid = sid * _NC + cid
    b = wid // _JPB
    j = wid % _JPB
    row0 = b * _S + _HEAD_S + j * _RPW

    sems = (sem0, sem1)
    lane_iota = lax.iota(jnp.int32, 16)

    def start(ci, slot):
        base = row0 + ci * _CH
        for g in range(_CH // 16):
            idxv[slot, pl.ds(16 * g, 16)] = base + 16 * g + lane_iota
        return pltpu.async_copy(
            x_hbm.at[idxv.at[slot]], buf.at[slot], sems[slot])

    for k in range(_LANES):
        accv[pl.ds(16 * k, 16)] = jnp.zeros((16,), jnp.float32)

    handles = {0: start(0, 0)}
    for ci in range(_NCH):
        slot = ci % 2
        if ci + 1 < _NCH:
            handles[(ci + 1) % 2] = start(ci + 1, (ci + 1) % 2)
        handles[slot].wait()

        for half in range(2):

            def row_body(r, a):
                return tuple(
                    a[k] + buf[slot, r, pl.ds(16 * (half * _HLANES + k), 16)]
                    for k in range(_HLANES))

            acc = lax.fori_loop(
                0, _CH, row_body,
                tuple(jnp.zeros((16,), jnp.float32)
                      for _ in range(_HLANES)))
            for k in range(_HLANES):
                kk = half * _HLANES + k
                accv[pl.ds(16 * kk, 16)] = accv[pl.ds(16 * kk, 16)] + acc[k]
    pltpu.sync_copy(accv, out_hbm.at[wid])


def _finalize_kernel(head_ref, parts_ref, wh_ref, bh_ref, wo_ref, bo_ref,
                     w_out_ref, i_out_ref):
    total = head_ref[...] + jnp.sum(parts_ref[...], axis=1)
    r = total * (1.0 / _S)
    h = r @ wh_ref[...] + bh_ref[...]
    h = h * jax.nn.sigmoid(h)
    logits = h @ wo_ref[...] + bo_ref[...]

    iota = jax.lax.broadcasted_iota(jnp.int32, logits.shape, 1)
    m1 = jnp.max(logits, axis=-1, keepdims=True)
    i1 = jnp.min(jnp.where(logits == m1, iota, _NUM_EXPERTS),
                 axis=-1, keepdims=True)
    masked = jnp.where(iota == i1, -jnp.inf, logits)
    m2 = jnp.max(masked, axis=-1, keepdims=True)
    i2 = jnp.min(jnp.where(masked == m2, iota, _NUM_EXPERTS),
                 axis=-1, keepdims=True)

    e2 = jnp.exp(m2 - m1)
    denom = 1.0 + e2
    w_out_ref[...] = jnp.concatenate([1.0 / denom, e2 / denom], axis=-1)
    i_out_ref[...] = jnp.concatenate([i1, i2], axis=-1).astype(jnp.int32)


@jax.jit
def kernel(x, W_hidden, b_hidden, W_out, b_out):
    x2d = x.reshape(_B * _S, _D)
    parts = _sc_tail_kernel(x2d)

    head = pl.pallas_call(
        _tc_head_kernel,
        grid=(_N_BLK,),
        in_specs=[pl.BlockSpec((_B, _S_BLK, _D), lambda s: (0, s, 0))],
        out_specs=pl.BlockSpec((_B, _D), lambda s: (0, 0)),
        out_shape=jax.ShapeDtypeStruct((_B, _D), jnp.float32),
        compiler_params=pltpu.CompilerParams(
            dimension_semantics=("arbitrary",),
        ),
    )(x)

    w, i = pl.pallas_call(
        _finalize_kernel,
        out_shape=(
            jax.ShapeDtypeStruct((_B, _TOP_K), jnp.float32),
            jax.ShapeDtypeStruct((_B, _TOP_K), jnp.int32),
        ),
    )(head, parts.reshape(_B, _JPB, _D), W_hidden,
      b_hidden.reshape(1, _D), W_out, b_out.reshape(1, _NUM_EXPERTS))
    return w, i


# speedup vs baseline: 1.3277x; 1.3277x over previous
import functools

import jax
import jax.numpy as jnp
from jax.experimental import pallas as pl
from jax.experimental.pallas import tpu as pltpu

_NUM_EXPERTS = 64
_TOP_K = 2
_B, _S, _D = 4, 8192, 768
_S_BLK = 256
_N_BLK = _S // _S_BLK


def _gate_kernel(x_ref, wh_ref, bh_ref, wo_ref, bo_ref,
                 w_out_ref, i_out_ref, acc_ref):
    step = pl.program_id(0)

    @pl.when(step == 0)
    def _init():
        acc_ref[...] = jnp.zeros_like(acc_ref)

    acc_ref[...] += jnp.sum(x_ref[...], axis=1)

    @pl.when(step == _N_BLK - 1)
    def _final():
        r = acc_ref[...] * (1.0 / _S)
        h = r @ wh_ref[...] + bh_ref[...]
        h = h * jax.nn.sigmoid(h)
        logits = h @ wo_ref[...] + bo_ref[...]

        iota = jax.lax.broadcasted_iota(jnp.int32, logits.shape, 1)
        m1 = jnp.max(logits, axis=-1, keepdims=True)
        i1 = jnp.min(jnp.where(logits == m1, iota, _NUM_EXPERTS),
                     axis=-1, keepdims=True)
        masked = jnp.where(iota == i1, -jnp.inf, logits)
        m2 = jnp.max(masked, axis=-1, keepdims=True)
        i2 = jnp.min(jnp.where(masked == m2, iota, _NUM_EXPERTS),
                     axis=-1, keepdims=True)

        e2 = jnp.exp(m2 - m1)
        denom = 1.0 + e2
        w_out_ref[...] = jnp.concatenate([1.0 / denom, e2 / denom], axis=-1)
        i_out_ref[...] = jnp.concatenate([i1, i2], axis=-1).astype(jnp.int32)


@jax.jit
def kernel(x, W_hidden, b_hidden, W_out, b_out):
    out_shapes = (
        jax.ShapeDtypeStruct((_B, _TOP_K), jnp.float32),
        jax.ShapeDtypeStruct((_B, _TOP_K), jnp.int32),
    )
    grid = (_N_BLK,)
    w, i = pl.pallas_call(
        _gate_kernel,
        grid=grid,
        in_specs=[
            pl.BlockSpec((_B, _S_BLK, _D), lambda s: (0, s, 0)),
            pl.BlockSpec((_D, _D), lambda s: (0, 0)),
            pl.BlockSpec((1, _D), lambda s: (0, 0)),
            pl.BlockSpec((_D, _NUM_EXPERTS), lambda s: (0, 0)),
            pl.BlockSpec((1, _NUM_EXPERTS), lambda s: (0, 0)),
        ],
        out_specs=(
            pl.BlockSpec((_B, _TOP_K), lambda s: (0, 0)),
            pl.BlockSpec((_B, _TOP_K), lambda s: (0, 0)),
        ),
        out_shape=out_shapes,
        scratch_shapes=[pltpu.VMEM((_B, _D), jnp.float32)],
        compiler_params=pltpu.CompilerParams(
            dimension_semantics=("arbitrary",),
        ),
    )(x, W_hidden, b_hidden.reshape(1, _D), W_out,
      b_out.reshape(1, _NUM_EXPERTS))
    return w, i


# speedup vs baseline: 1.5124x; 1.1391x over previous
import functools

import jax
import jax.numpy as jnp
from jax.experimental import pallas as pl
from jax.experimental.pallas import tpu as pltpu

_NUM_EXPERTS = 64
_TOP_K = 2
_B, _S, _D = 4, 8192, 768
_S_BLK = 512
_N_STREAMS = 2
_N_BLK = _S // (_S_BLK * _N_STREAMS)


def _gate_kernel(xa_ref, xb_ref, wh_ref, bh_ref, wo_ref, bo_ref,
                 w_out_ref, i_out_ref, acc_ref):
    step = pl.program_id(0)

    @pl.when(step == 0)
    def _init():
        acc_ref[...] = jnp.zeros_like(acc_ref)

    acc_ref[...] += jnp.sum(xa_ref[...], axis=1) + jnp.sum(xb_ref[...], axis=1)

    @pl.when(step == _N_BLK - 1)
    def _final():
        r = acc_ref[...] * (1.0 / _S)
        h = r @ wh_ref[...] + bh_ref[...]
        h = h * jax.nn.sigmoid(h)
        logits = h @ wo_ref[...] + bo_ref[...]

        iota = jax.lax.broadcasted_iota(jnp.int32, logits.shape, 1)
        m1 = jnp.max(logits, axis=-1, keepdims=True)
        i1 = jnp.min(jnp.where(logits == m1, iota, _NUM_EXPERTS),
                     axis=-1, keepdims=True)
        masked = jnp.where(iota == i1, -jnp.inf, logits)
        m2 = jnp.max(masked, axis=-1, keepdims=True)
        i2 = jnp.min(jnp.where(masked == m2, iota, _NUM_EXPERTS),
                     axis=-1, keepdims=True)

        e2 = jnp.exp(m2 - m1)
        denom = 1.0 + e2
        w_out_ref[...] = jnp.concatenate([1.0 / denom, e2 / denom], axis=-1)
        i_out_ref[...] = jnp.concatenate([i1, i2], axis=-1).astype(jnp.int32)


@jax.jit
def kernel(x, W_hidden, b_hidden, W_out, b_out):
    out_shapes = (
        jax.ShapeDtypeStruct((_B, _TOP_K), jnp.float32),
        jax.ShapeDtypeStruct((_B, _TOP_K), jnp.int32),
    )
    grid = (_N_BLK,)
    w, i = pl.pallas_call(
        _gate_kernel,
        grid=grid,
        in_specs=[
            pl.BlockSpec((_B, _S_BLK, _D), lambda s: (0, s, 0)),
            pl.BlockSpec((_B, _S_BLK, _D), lambda s: (0, s + _N_BLK, 0)),
            pl.BlockSpec((_D, _D), lambda s: (0, 0)),
            pl.BlockSpec((1, _D), lambda s: (0, 0)),
            pl.BlockSpec((_D, _NUM_EXPERTS), lambda s: (0, 0)),
            pl.BlockSpec((1, _NUM_EXPERTS), lambda s: (0, 0)),
        ],
        out_specs=(
            pl.BlockSpec((_B, _TOP_K), lambda s: (0, 0)),
            pl.BlockSpec((_B, _TOP_K), lambda s: (0, 0)),
        ),
        out_shape=out_shapes,
        scratch_shapes=[pltpu.VMEM((_B, _D), jnp.float32)],
        compiler_params=pltpu.CompilerParams(
            dimension_semantics=("arbitrary",),
        ),
    )(x, x, W_hidden, b_hidden.reshape(1, _D), W_out,
      b_out.reshape(1, _NUM_EXPERTS))
    return w, i


# speedup vs baseline: 1.5131x; 1.0005x over previous
import functools

import jax
import jax.numpy as jnp
from jax.experimental import pallas as pl
from jax.experimental.pallas import tpu as pltpu

_NUM_EXPERTS = 64
_TOP_K = 2
_B, _S, _D = 4, 8192, 768
_S_BLK = 512
_N_BLK = _S // _S_BLK


def _gate_kernel(x_ref, wh_ref, bh_ref, wo_ref, bo_ref,
                 w_out_ref, i_out_ref, acc_ref):
    step = pl.program_id(0)

    @pl.when(step == 0)
    def _init():
        acc_ref[...] = jnp.zeros_like(acc_ref)

    ones = jnp.ones((8, _S_BLK), jnp.float32)
    for b in range(_B):
        partial = jax.lax.dot_general(
            ones, x_ref[b], (((1,), (0,)), ((), ())),
            preferred_element_type=jnp.float32)
        acc_ref[pl.ds(b, 1), :] += partial[0:1, :]

    @pl.when(step == _N_BLK - 1)
    def _final():
        r = acc_ref[...] * (1.0 / _S)
        h = r @ wh_ref[...] + bh_ref[...]
        h = h * jax.nn.sigmoid(h)
        logits = h @ wo_ref[...] + bo_ref[...]

        iota = jax.lax.broadcasted_iota(jnp.int32, logits.shape, 1)
        m1 = jnp.max(logits, axis=-1, keepdims=True)
        i1 = jnp.min(jnp.where(logits == m1, iota, _NUM_EXPERTS),
                     axis=-1, keepdims=True)
        masked = jnp.where(iota == i1, -jnp.inf, logits)
        m2 = jnp.max(masked, axis=-1, keepdims=True)
        i2 = jnp.min(jnp.where(masked == m2, iota, _NUM_EXPERTS),
                     axis=-1, keepdims=True)

        e2 = jnp.exp(m2 - m1)
        denom = 1.0 + e2
        w_out_ref[...] = jnp.concatenate([1.0 / denom, e2 / denom], axis=-1)
        i_out_ref[...] = jnp.concatenate([i1, i2], axis=-1).astype(jnp.int32)


@jax.jit
def kernel(x, W_hidden, b_hidden, W_out, b_out):
    out_shapes = (
        jax.ShapeDtypeStruct((_B, _TOP_K), jnp.float32),
        jax.ShapeDtypeStruct((_B, _TOP_K), jnp.int32),
    )
    grid = (_N_BLK,)
    w, i = pl.pallas_call(
        _gate_kernel,
        grid=grid,
        in_specs=[
            pl.BlockSpec((_B, _S_BLK, _D), lambda s: (0, s, 0)),
            pl.BlockSpec((_D, _D), lambda s: (0, 0)),
            pl.BlockSpec((1, _D), lambda s: (0, 0)),
            pl.BlockSpec((_D, _NUM_EXPERTS), lambda s: (0, 0)),
            pl.BlockSpec((1, _NUM_EXPERTS), lambda s: (0, 0)),
        ],
        out_specs=(
            pl.BlockSpec((_B, _TOP_K), lambda s: (0, 0)),
            pl.BlockSpec((_B, _TOP_K), lambda s: (0, 0)),
        ),
        out_shape=out_shapes,
        scratch_shapes=[pltpu.VMEM((_B, _D), jnp.float32)],
        compiler_params=pltpu.CompilerParams(
            dimension_semantics=("arbitrary",),
        ),
    )(x, W_hidden, b_hidden.reshape(1, _D), W_out,
      b_out.reshape(1, _NUM_EXPERTS))
    return w, i


# speedup vs baseline: 1.5146x; 1.0010x over previous
import functools

import jax
import jax.numpy as jnp
from jax.experimental import pallas as pl
from jax.experimental.pallas import tpu as pltpu

_NUM_EXPERTS = 64
_TOP_K = 2
_B, _S, _D = 4, 8192, 768
_R_BLK = 2048
_N_BLK = _B * _S // _R_BLK
_BPB = _S // _R_BLK


def _gate_kernel(x_ref, wh_ref, bh_ref, wo_ref, bo_ref,
                 w_out_ref, i_out_ref, acc_ref):
    step = pl.program_id(0)
    b = step // _BPB

    @pl.when(step == 0)
    def _init():
        acc_ref[...] = jnp.zeros_like(acc_ref)

    s = jnp.sum(x_ref[...], axis=0)
    row = jax.lax.broadcasted_iota(jnp.int32, (8, _D), 0)
    acc_ref[...] += jnp.where(row == b, s[None, :], 0.0)

    @pl.when(step == _N_BLK - 1)
    def _final():
        r = acc_ref[0:_B, :] * (1.0 / _S)
        h = r @ wh_ref[...] + bh_ref[...]
        h = h * jax.nn.sigmoid(h)
        logits = h @ wo_ref[...] + bo_ref[...]

        iota = jax.lax.broadcasted_iota(jnp.int32, logits.shape, 1)
        m1 = jnp.max(logits, axis=-1, keepdims=True)
        i1 = jnp.min(jnp.where(logits == m1, iota, _NUM_EXPERTS),
                     axis=-1, keepdims=True)
        masked = jnp.where(iota == i1, -jnp.inf, logits)
        m2 = jnp.max(masked, axis=-1, keepdims=True)
        i2 = jnp.min(jnp.where(masked == m2, iota, _NUM_EXPERTS),
                     axis=-1, keepdims=True)

        e2 = jnp.exp(m2 - m1)
        denom = 1.0 + e2
        w_out_ref[...] = jnp.concatenate([1.0 / denom, e2 / denom], axis=-1)
        i_out_ref[...] = jnp.concatenate([i1, i2], axis=-1).astype(jnp.int32)


@jax.jit
def kernel(x, W_hidden, b_hidden, W_out, b_out):
    x2d = x.reshape(_B * _S, _D)
    out_shapes = (
        jax.ShapeDtypeStruct((_B, _TOP_K), jnp.float32),
        jax.ShapeDtypeStruct((_B, _TOP_K), jnp.int32),
    )
    w, i = pl.pallas_call(
        _gate_kernel,
        grid=(_N_BLK,),
        in_specs=[
            pl.BlockSpec((_R_BLK, _D), lambda s: (s, 0)),
            pl.BlockSpec((_D, _D), lambda s: (0, 0)),
            pl.BlockSpec((1, _D), lambda s: (0, 0)),
            pl.BlockSpec((_D, _NUM_EXPERTS), lambda s: (0, 0)),
            pl.BlockSpec((1, _NUM_EXPERTS), lambda s: (0, 0)),
        ],
        out_specs=(
            pl.BlockSpec((_B, _TOP_K), lambda s: (0, 0)),
            pl.BlockSpec((_B, _TOP_K), lambda s: (0, 0)),
        ),
        out_shape=out_shapes,
        scratch_shapes=[pltpu.VMEM((8, _D), jnp.float32)],
        compiler_params=pltpu.CompilerParams(
            dimension_semantics=("arbitrary",),
        ),
    )(x2d, W_hidden, b_hidden.reshape(1, _D), W_out,
      b_out.reshape(1, _NUM_EXPERTS))
    return w, i


# speedup vs baseline: 1.5275x; 1.0085x over previous
import functools

import jax
import jax.numpy as jnp
from jax.experimental import pallas as pl
from jax.experimental.pallas import tpu as pltpu

_NUM_EXPERTS = 64
_TOP_K = 2
_B, _S, _D = 4, 8192, 768
_R_BLK = 4096
_N_BLK = _B * _S // _R_BLK
_BPB = _S // _R_BLK


def _gate_kernel(x_ref, wh_ref, bh_ref, wo_ref, bo_ref,
                 w_out_ref, i_out_ref, acc_ref):
    step = pl.program_id(0)
    b = step // _BPB

    @pl.when(step == 0)
    def _init():
        acc_ref[...] = jnp.zeros_like(acc_ref)

    s = jnp.sum(x_ref[...], axis=0)
    row = jax.lax.broadcasted_iota(jnp.int32, (8, _D), 0)
    acc_ref[...] += jnp.where(row == b, s[None, :], 0.0)

    @pl.when(step == _N_BLK - 1)
    def _final():
        r = acc_ref[0:_B, :] * (1.0 / _S)
        h = r @ wh_ref[...] + bh_ref[...]
        h = h * jax.nn.sigmoid(h)
        logits = h @ wo_ref[...] + bo_ref[...]

        iota = jax.lax.broadcasted_iota(jnp.int32, logits.shape, 1)
        m1 = jnp.max(logits, axis=-1, keepdims=True)
        i1 = jnp.min(jnp.where(logits == m1, iota, _NUM_EXPERTS),
                     axis=-1, keepdims=True)
        masked = jnp.where(iota == i1, -jnp.inf, logits)
        m2 = jnp.max(masked, axis=-1, keepdims=True)
        i2 = jnp.min(jnp.where(masked == m2, iota, _NUM_EXPERTS),
                     axis=-1, keepdims=True)

        e2 = jnp.exp(m2 - m1)
        denom = 1.0 + e2
        w_out_ref[...] = jnp.concatenate([1.0 / denom, e2 / denom], axis=-1)
        i_out_ref[...] = jnp.concatenate([i1, i2], axis=-1).astype(jnp.int32)


@jax.jit
def kernel(x, W_hidden, b_hidden, W_out, b_out):
    x2d = x.reshape(_B * _S, _D)
    out_shapes = (
        jax.ShapeDtypeStruct((_B, _TOP_K), jnp.float32),
        jax.ShapeDtypeStruct((_B, _TOP_K), jnp.int32),
    )
    w, i = pl.pallas_call(
        _gate_kernel,
        grid=(_N_BLK,),
        in_specs=[
            pl.BlockSpec((_R_BLK, _D), lambda s: (s, 0)),
            pl.BlockSpec((_D, _D), lambda s: (0, 0)),
            pl.BlockSpec((1, _D), lambda s: (0, 0)),
            pl.BlockSpec((_D, _NUM_EXPERTS), lambda s: (0, 0)),
            pl.BlockSpec((1, _NUM_EXPERTS), lambda s: (0, 0)),
        ],
        out_specs=(
            pl.BlockSpec((_B, _TOP_K), lambda s: (0, 0)),
            pl.BlockSpec((_B, _TOP_K), lambda s: (0, 0)),
        ),
        out_shape=out_shapes,
        scratch_shapes=[pltpu.VMEM((8, _D), jnp.float32)],
        compiler_params=pltpu.CompilerParams(
            dimension_semantics=("arbitrary",),
        ),
    )(x2d, W_hidden, b_hidden.reshape(1, _D), W_out,
      b_out.reshape(1, _NUM_EXPERTS))
    return w, i
